# baseline (speedup 1.0000x reference)
import jax
import jax.numpy as jnp
from jax.experimental import pallas as pl
from jax.experimental.pallas import tpu as pltpu

B, S, H, E, K, I = 1, 2048, 768, 8, 2, 3072
BALANCE_COEF = 0.01
N = B * S * K
TM = 512
NB = N // TM
G = NB + E - 1
J = 4
TI = I // J
LANES = 128


def _router_body(x_ref, wg_ref, idx_ref, w_ref, counts_ref, loss_ref):
    x = x_ref[...]
    wg = wg_ref[...]
    logits = jax.lax.dot_general(
        x, wg, (((1,), (0,)), ((), ())), preferred_element_type=jnp.float32)
    lane = jax.lax.broadcasted_iota(jnp.int32, (S, LANES), 1)
    valid = lane < E
    lg = jnp.where(valid, logits, -1e30)
    m = jnp.max(lg, axis=1, keepdims=True)
    p = jnp.where(valid, jnp.exp(lg - m), 0.0)
    probs = p / jnp.sum(p, axis=1, keepdims=True)
    v1 = jnp.max(probs, axis=1, keepdims=True)
    i1 = jnp.min(jnp.where(probs == v1, lane, LANES), axis=1, keepdims=True)
    probs_m = jnp.where(lane == i1, -1.0, probs)
    v2 = jnp.max(probs_m, axis=1, keepdims=True)
    i2 = jnp.min(jnp.where(probs_m == v2, lane, LANES), axis=1, keepdims=True)
    denom = v1 + v2
    idx_ref[...] = jnp.where(lane == 0, i1,
                             jnp.where(lane == 1, i2, 0)).astype(jnp.int32)
    w_ref[...] = jnp.where(lane == 0, v1 / denom,
                           jnp.where(lane == 1, v2 / denom, 0.0))
    onehot = ((lane == i1) | (lane == i2)).astype(jnp.float32)
    counts = jnp.sum(onehot, axis=0, keepdims=True)
    counts_ref[...] = counts.astype(jnp.int32)
    pmean = jnp.mean(probs, axis=0, keepdims=True)
    f = counts / jnp.float32(S)
    loss = BALANCE_COEF * E * jnp.sum(f * pmean)
    lane0 = jax.lax.broadcasted_iota(jnp.int32, (1, LANES), 1)
    loss_ref[...] = jnp.where(lane0 == 0, loss, 0.0)


def _router(x, wg_padded):
    return pl.pallas_call(
        _router_body,
        out_shape=(
            jax.ShapeDtypeStruct((S, LANES), jnp.int32),
            jax.ShapeDtypeStruct((S, LANES), jnp.float32),
            jax.ShapeDtypeStruct((1, LANES), jnp.int32),
            jax.ShapeDtypeStruct((1, LANES), jnp.float32),
        ),
    )(x, wg_padded)


def _ffn_body(cuts_ref, blk_ref, exp_ref, isf_ref, isl_ref,
              x_ref, *rest):
    w1_refs = rest[:J]
    b1_ref = rest[J]
    w2_refs = rest[J + 1:2 * J + 1]
    b2_ref = rest[2 * J + 1]
    ws_ref = rest[2 * J + 2]
    out_ref = rest[2 * J + 3]
    acc_ref = rest[2 * J + 4]
    g = pl.program_id(0)

    @pl.when(isf_ref[g] == 1)
    def _():
        acc_ref[...] = jnp.zeros_like(acc_ref)

    x = x_ref[...].astype(jnp.bfloat16)
    y = b2_ref[0]
    for j in range(J):
        hj = jnp.dot(x, w1_refs[j][0].astype(jnp.bfloat16),
                     preferred_element_type=jnp.float32)
        hj = hj + b1_ref[0, :, pl.ds(j * TI, TI)]
        hj = jax.nn.gelu(hj).astype(jnp.bfloat16)
        y = y + jnp.dot(hj, w2_refs[j][0].astype(jnp.bfloat16),
                        preferred_element_type=jnp.float32)
    row = blk_ref[g] * TM + jax.lax.broadcasted_iota(jnp.int32, (TM, 1), 0)
    mask = (row >= cuts_ref[g]) & (row < cuts_ref[g + 1])
    mw = jnp.where(mask, ws_ref[...], 0.0)
    acc_ref[...] += mw * y

    @pl.when(isl_ref[g] == 1)
    def _():
        out_ref[...] = acc_ref[...]


def _grouped_ffn(cuts, blk_ids, exp_ids, isf, isl, x_sorted, W1, b1r, W2, b2r, ws2d):
    w1_specs = [
        pl.BlockSpec((1, H, TI), lambda g, c, b, e, f, l, j=j: (e[g], 0, j))
        for j in range(J)
    ]
    w2_specs = [
        pl.BlockSpec((1, TI, H), lambda g, c, b, e, f, l, j=j: (e[g], j, 0))
        for j in range(J)
    ]
    grid_spec = pltpu.PrefetchScalarGridSpec(
        num_scalar_prefetch=5,
        grid=(G,),
        in_specs=[
            pl.BlockSpec((TM, H), lambda g, c, b, e, f, l: (b[g], 0)),
            *w1_specs,
            pl.BlockSpec((1, 1, I), lambda g, c, b, e, f, l: (e[g], 0, 0)),
            *w2_specs,
            pl.BlockSpec((1, 1, H), lambda g, c, b, e, f, l: (e[g], 0, 0)),
            pl.BlockSpec((TM, 1), lambda g, c, b, e, f, l: (b[g], 0)),
        ],
        out_specs=pl.BlockSpec((TM, H), lambda g, c, b, e, f, l: (b[g], 0)),
        scratch_shapes=[pltpu.VMEM((TM, H), jnp.float32)],
    )
    return pl.pallas_call(
        _ffn_body,
        grid_spec=grid_spec,
        out_shape=jax.ShapeDtypeStruct((N, H), jnp.float32),
    )(cuts, blk_ids, exp_ids, isf, isl,
      x_sorted, *([W1] * J), b1r, *([W2] * J), b2r, ws2d)


def kernel(hidden_states, Wg, W1, b1, W2, b2):
    x = hidden_states.reshape(S, H)
    wg_padded = jnp.pad(Wg, ((0, 0), (0, LANES - E)))

    idx_out, w_out, counts_out, loss_out = _router(x, wg_padded)
    balance_loss = loss_out[0, 0]
    counts = counts_out[0, :E]
    experts_flat = idx_out[:, :K].reshape(-1)
    weights_flat = w_out[:, :K].reshape(-1)

    offs = jnp.concatenate([jnp.zeros((1,), jnp.int32),
                            jnp.cumsum(counts, dtype=jnp.int32)])
    sort_idx = jnp.argsort(experts_flat).astype(jnp.int32)
    x_sorted = jnp.take(x, sort_idx // K, axis=0)
    ws2d = jnp.take(weights_flat, sort_idx)[:, None]

    blk_bounds = jnp.arange(NB, dtype=jnp.int32) * TM
    cuts = jnp.sort(jnp.concatenate([blk_bounds, offs[1:E]]))
    cuts_full = jnp.concatenate([cuts, jnp.full((1,), N, jnp.int32)])
    blk_ids = jnp.clip(cuts // TM, 0, NB - 1).astype(jnp.int32)
    exp_ids = jnp.clip(jnp.searchsorted(offs, cuts, side="right") - 1,
                       0, E - 1).astype(jnp.int32)
    prev = jnp.concatenate([jnp.full((1,), -1, jnp.int32), blk_ids[:-1]])
    nxt = jnp.concatenate([blk_ids[1:], jnp.full((1,), -1, jnp.int32)])
    isf = (blk_ids != prev).astype(jnp.int32)
    isl = (blk_ids != nxt).astype(jnp.int32)

    b1r = b1[:, None, :]
    b2r = b2[:, None, :]
    y_sorted = _grouped_ffn(cuts_full, blk_ids, exp_ids, isf, isl,
                            x_sorted, W1, b1r, W2, b2r, ws2d)

    inv = jnp.argsort(sort_idx).astype(jnp.int32)
    y_pairs = jnp.take(y_sorted, inv, axis=0).reshape(S, K, H)
    out = (x + y_pairs.sum(axis=1)).reshape(B, S, H)
    return out, balance_loss


# speedup vs baseline: 1.0176x; 1.0176x over previous
import jax
import jax.numpy as jnp
from jax.experimental import pallas as pl
from jax.experimental.pallas import tpu as pltpu

B, S, H, E, K, I = 1, 2048, 768, 8, 2, 3072
BALANCE_COEF = 0.01
N = B * S * K
TM = 512
NB = N // TM
G = NB + E - 1
J = 4
TI = I // J
LANES = 128


def _router_body(x_ref, wg_ref, idx_ref, w_ref, loss_ref,
                 cuts_ref2, blk_ref2, exp_ref2, isf_ref2, isl_ref2):
    x = x_ref[...]
    wg = wg_ref[...]
    logits = jax.lax.dot_general(
        x, wg, (((1,), (0,)), ((), ())), preferred_element_type=jnp.float32)
    lane = jax.lax.broadcasted_iota(jnp.int32, (S, LANES), 1)
    valid = lane < E
    lg = jnp.where(valid, logits, -1e30)
    m = jnp.max(lg, axis=1, keepdims=True)
    p = jnp.where(valid, jnp.exp(lg - m), 0.0)
    probs = p / jnp.sum(p, axis=1, keepdims=True)
    v1 = jnp.max(probs, axis=1, keepdims=True)
    i1 = jnp.min(jnp.where(probs == v1, lane, LANES), axis=1, keepdims=True)
    probs_m = jnp.where(lane == i1, -1.0, probs)
    v2 = jnp.max(probs_m, axis=1, keepdims=True)
    i2 = jnp.min(jnp.where(probs_m == v2, lane, LANES), axis=1, keepdims=True)
    denom = v1 + v2
    idx_ref[...] = jnp.where(lane == 0, i1,
                             jnp.where(lane == 1, i2, 0)).astype(jnp.int32)
    w_ref[...] = jnp.where(lane == 0, v1 / denom,
                           jnp.where(lane == 1, v2 / denom, 0.0))
    onehot = ((lane == i1) | (lane == i2)).astype(jnp.float32)
    counts = jnp.sum(onehot, axis=0, keepdims=True)
    pmean = jnp.mean(probs, axis=0, keepdims=True)
    f = counts / jnp.float32(S)
    loss = BALANCE_COEF * E * jnp.sum(f * pmean)
    lane0 = jax.lax.broadcasted_iota(jnp.int32, (1, LANES), 1)
    loss_ref[...] = jnp.where(lane0 == 0, loss, 0.0)

    ri = jax.lax.broadcasted_iota(jnp.int32, (LANES, LANES), 0)
    ci = jax.lax.broadcasted_iota(jnp.int32, (LANES, LANES), 1)
    lane_r = lane0
    row_l = jax.lax.broadcasted_iota(jnp.int32, (LANES, 1), 0)
    lt = (ri < ci).astype(jnp.float32)
    offs_row = jax.lax.dot_general(
        counts, lt, (((1,), (0,)), ((), ())),
        preferred_element_type=jnp.float32)
    offs_bcast = jnp.broadcast_to(offs_row, (LANES, LANES))
    offs_col = jnp.sum(offs_bcast * (ri == ci), axis=1, keepdims=True)
    sh = ((ri + (NB - 1) == ci) & (ri >= 1) & (ri <= E - 1)).astype(jnp.float32)
    voff_row = jax.lax.dot_general(
        offs_row, sh, (((1,), (0,)), ((), ())),
        preferred_element_type=jnp.float32)
    m3 = (ci + (NB - 1) == ri).astype(jnp.float32)
    voff_col = jnp.sum(offs_bcast * m3, axis=1, keepdims=True)
    big = jnp.float32(1 << 20)
    v_row = jnp.where(lane_r < NB, lane_r.astype(jnp.float32) * TM,
                      jnp.where(lane_r < G, voff_row, big))
    v_col = jnp.where(row_l < NB, row_l.astype(jnp.float32) * TM,
                      jnp.where(row_l < G, voff_col, big))
    vb_c = jnp.broadcast_to(v_row, (LANES, LANES))
    vb_r = jnp.broadcast_to(v_col, (LANES, LANES))
    before = (vb_c < vb_r) | ((vb_c == vb_r) & (ci < ri))
    rank_col = jnp.sum(before.astype(jnp.float32), axis=1, keepdims=True)
    scat = (rank_col == ci.astype(jnp.float32)).astype(jnp.float32)
    cuts_f = jax.lax.dot_general(
        v_row, scat, (((1,), (0,)), ((), ())),
        preferred_element_type=jnp.float32)
    cuts_i = jnp.where(lane_r >= G, N, cuts_f.astype(jnp.int32))
    cuts_ref2[...] = cuts_i
    blk_ref2[...] = jnp.clip(cuts_i // TM, 0, NB - 1)
    in_e = (jnp.broadcast_to(offs_col, (LANES, LANES)) <=
            jnp.broadcast_to(cuts_f, (LANES, LANES))) & (ri <= E)
    exp_cnt = jnp.sum(in_e.astype(jnp.float32), axis=0, keepdims=True)
    exp_ref2[...] = jnp.clip(exp_cnt.astype(jnp.int32) - 1, 0, E - 1)
    isf_ref2[...] = (jax.lax.rem(cuts_i, TM) == 0).astype(jnp.int32)
    shl = (ri == ci + 1).astype(jnp.float32)
    nxt_f = jax.lax.dot_general(
        cuts_i.astype(jnp.float32), shl, (((1,), (0,)), ((), ())),
        preferred_element_type=jnp.float32)
    isl_ref2[...] = (jax.lax.rem(nxt_f.astype(jnp.int32), TM) == 0).astype(jnp.int32)


def _router(x, wg_padded):
    return pl.pallas_call(
        _router_body,
        out_shape=(
            jax.ShapeDtypeStruct((S, LANES), jnp.int32),
            jax.ShapeDtypeStruct((S, LANES), jnp.float32),
            jax.ShapeDtypeStruct((1, LANES), jnp.float32),
            jax.ShapeDtypeStruct((1, LANES), jnp.int32),
            jax.ShapeDtypeStruct((1, LANES), jnp.int32),
            jax.ShapeDtypeStruct((1, LANES), jnp.int32),
            jax.ShapeDtypeStruct((1, LANES), jnp.int32),
            jax.ShapeDtypeStruct((1, LANES), jnp.int32),
        ),
    )(x, wg_padded)


def _ffn_body(cuts_ref, blk_ref, exp_ref, isf_ref, isl_ref,
              x_ref, *rest):
    w1_refs = rest[:J]
    b1_ref = rest[J]
    w2_refs = rest[J + 1:2 * J + 1]
    b2_ref = rest[2 * J + 1]
    ws_ref = rest[2 * J + 2]
    out_ref = rest[2 * J + 3]
    acc_ref = rest[2 * J + 4]
    g = pl.program_id(0)

    @pl.when(isf_ref[0, g] == 1)
    def _():
        acc_ref[...] = jnp.zeros_like(acc_ref)

    x = x_ref[...].astype(jnp.bfloat16)
    y = b2_ref[0]
    for j in range(J):
        hj = jnp.dot(x, w1_refs[j][0].astype(jnp.bfloat16),
                     preferred_element_type=jnp.float32)
        hj = hj + b1_ref[0, :, pl.ds(j * TI, TI)]
        hj = jax.nn.gelu(hj).astype(jnp.bfloat16)
        y = y + jnp.dot(hj, w2_refs[j][0].astype(jnp.bfloat16),
                        preferred_element_type=jnp.float32)
    row = blk_ref[0, g] * TM + jax.lax.broadcasted_iota(jnp.int32, (TM, 1), 0)
    mask = (row >= cuts_ref[0, g]) & (row < cuts_ref[0, g + 1])
    mw = jnp.where(mask, ws_ref[...], 0.0)
    acc_ref[...] += mw * y

    @pl.when(isl_ref[0, g] == 1)
    def _():
        out_ref[...] = acc_ref[...]


def _grouped_ffn(cuts, blk_ids, exp_ids, isf, isl, x_sorted, W1, b1r, W2, b2r, ws2d):
    w1_specs = [
        pl.BlockSpec((1, H, TI), lambda g, c, b, e, f, l, j=j: (e[0, g], 0, j))
        for j in range(J)
    ]
    w2_specs = [
        pl.BlockSpec((1, TI, H), lambda g, c, b, e, f, l, j=j: (e[0, g], j, 0))
        for j in range(J)
    ]
    grid_spec = pltpu.PrefetchScalarGridSpec(
        num_scalar_prefetch=5,
        grid=(G,),
        in_specs=[
            pl.BlockSpec((TM, H), lambda g, c, b, e, f, l: (b[0, g], 0)),
            *w1_specs,
            pl.BlockSpec((1, 1, I), lambda g, c, b, e, f, l: (e[0, g], 0, 0)),
            *w2_specs,
            pl.BlockSpec((1, 1, H), lambda g, c, b, e, f, l: (e[0, g], 0, 0)),
            pl.BlockSpec((TM, 1), lambda g, c, b, e, f, l: (b[0, g], 0)),
        ],
        out_specs=pl.BlockSpec((TM, H), lambda g, c, b, e, f, l: (b[0, g], 0)),
        scratch_shapes=[pltpu.VMEM((TM, H), jnp.float32)],
    )
    return pl.pallas_call(
        _ffn_body,
        grid_spec=grid_spec,
        out_shape=jax.ShapeDtypeStruct((N, H), jnp.float32),
    )(cuts, blk_ids, exp_ids, isf, isl,
      x_sorted, *([W1] * J), b1r, *([W2] * J), b2r, ws2d)


def kernel(hidden_states, Wg, W1, b1, W2, b2):
    x = hidden_states.reshape(S, H)
    wg_padded = jnp.pad(Wg, ((0, 0), (0, LANES - E)))

    (idx_out, w_out, loss_out,
     cuts_full, blk_ids, exp_ids, isf, isl) = _router(x, wg_padded)
    balance_loss = loss_out[0, 0]
    experts_flat = idx_out[:, :K].reshape(-1)
    weights_flat = w_out[:, :K].reshape(-1)

    sort_idx = jnp.argsort(experts_flat).astype(jnp.int32)
    x_sorted = jnp.take(x, sort_idx // K, axis=0)
    ws2d = jnp.take(weights_flat, sort_idx)[:, None]

    b1r = b1[:, None, :]
    b2r = b2[:, None, :]
    y_sorted = _grouped_ffn(cuts_full, blk_ids, exp_ids, isf, isl,
                            x_sorted, W1, b1r, W2, b2r, ws2d)

    inv = jnp.argsort(sort_idx).astype(jnp.int32)
    y_pairs = jnp.take(y_sorted, inv, axis=0).reshape(S, K, H)
    out = (x + y_pairs.sum(axis=1)).reshape(B, S, H)
    return out, balance_loss
